# baseline (speedup 1.0000x reference)
import functools

import jax
import jax.numpy as jnp
from jax import lax
from jax.experimental import pallas as pl
from jax.experimental.pallas import tpu as pltpu
from jax.experimental.pallas import tpu_sc as plsc


def _project_body(cbfv_ref, w_ref, b_ref, out_ref):
    acc = lax.dot_general(
        cbfv_ref[...], w_ref[...],
        dimension_numbers=(((1,), (1,)), ((), ())),
        preferred_element_type=jnp.float32,
    )
    out_ref[...] = acc + b_ref[...][None, :]


def _project_table(cbfv, W, b):
    vocab = cbfv.shape[0]
    d_model = W.shape[0]
    return pl.pallas_call(
        _project_body,
        out_shape=jax.ShapeDtypeStruct((vocab, d_model), jnp.float32),
    )(cbfv, W, b)


_CHUNK = 128


@functools.partial(jax.jit, static_argnums=(2, 3))
def _sc_gather(table, idx, n_rows, d_model):
    try:
        info = plsc.get_sparse_core_info()
        nc, ns = info.num_cores, info.num_subcores
    except Exception:
        nc, ns = 2, 16
    nw = nc * ns
    assert n_rows % (nw * _CHUNK) == 0
    rows_per_w = n_rows // nw
    n_chunks = rows_per_w // _CHUNK

    mesh = plsc.VectorSubcoreMesh(core_axis_name="c", subcore_axis_name="s")

    @functools.partial(
        pl.kernel,
        mesh=mesh,
        out_type=jax.ShapeDtypeStruct((n_rows, d_model), jnp.float32),
        scratch_types=[
            pltpu.VMEM((rows_per_w,), jnp.int32),
            pltpu.VMEM((_CHUNK, d_model), jnp.float32),
            pltpu.SemaphoreType.DMA,
        ],
    )
    def gather_kernel(table_hbm, idx_hbm, out_hbm, idx_v, rows_v, sem):
        wid = lax.axis_index("s") * nc + lax.axis_index("c")
        base = pl.multiple_of(wid * rows_per_w, rows_per_w)
        pltpu.sync_copy(idx_hbm.at[pl.ds(base, rows_per_w)], idx_v)

        def body(j, carry):
            off = pl.multiple_of(j * _CHUNK, _CHUNK)
            pltpu.async_copy(
                table_hbm.at[idx_v.at[pl.ds(off, _CHUNK)]], rows_v, sem
            ).wait()
            pltpu.sync_copy(rows_v, out_hbm.at[pl.ds(base + off, _CHUNK)])
            return carry

        lax.fori_loop(0, n_chunks, body, 0)

    return gather_kernel(table, idx)


def kernel(src, cbfv, W, b):
    batch, seq = src.shape
    d_model = W.shape[0]
    n_rows = batch * seq
    table = _project_table(cbfv, W, b)
    idx = src.reshape(-1).astype(jnp.int32)
    out = _sc_gather(table, idx, n_rows, d_model)
    return out.reshape(batch, seq, d_model)


# speedup vs baseline: 1.0023x; 1.0023x over previous
import functools

import jax
import jax.numpy as jnp
from jax import lax
from jax.experimental import pallas as pl
from jax.experimental.pallas import tpu as pltpu
from jax.experimental.pallas import tpu_sc as plsc


def _project_body(cbfv_ref, w_ref, b_ref, out_ref):
    acc = lax.dot_general(
        cbfv_ref[...], w_ref[...],
        dimension_numbers=(((1,), (1,)), ((), ())),
        preferred_element_type=jnp.float32,
    )
    out_ref[...] = acc + b_ref[...][None, :]


def _project_table(cbfv, W, b):
    vocab = cbfv.shape[0]
    d_model = W.shape[0]
    return pl.pallas_call(
        _project_body,
        out_shape=jax.ShapeDtypeStruct((vocab, d_model), jnp.float32),
    )(cbfv, W, b)


_CHUNK = 80


@functools.partial(jax.jit, static_argnums=(2, 3))
def _sc_gather(table, idx, n_rows, d_model):
    try:
        info = plsc.get_sparse_core_info()
        nc, ns = info.num_cores, info.num_subcores
    except Exception:
        nc, ns = 2, 16
    nw = nc * ns
    assert n_rows % (nw * _CHUNK) == 0
    rows_per_w = n_rows // nw
    n_chunks = rows_per_w // _CHUNK
    assert n_chunks % 2 == 0 and n_chunks >= 4

    mesh = plsc.VectorSubcoreMesh(core_axis_name="c", subcore_axis_name="s")

    @functools.partial(
        pl.kernel,
        mesh=mesh,
        out_type=jax.ShapeDtypeStruct((n_rows, d_model), jnp.float32),
        scratch_types=[
            pltpu.VMEM((rows_per_w,), jnp.int32),
            pltpu.VMEM((_CHUNK, d_model), jnp.float32),
            pltpu.VMEM((_CHUNK, d_model), jnp.float32),
            pltpu.SemaphoreType.DMA,
            pltpu.SemaphoreType.DMA,
            pltpu.SemaphoreType.DMA,
            pltpu.SemaphoreType.DMA,
        ],
    )
    def gather_kernel(table_hbm, idx_hbm, out_hbm, idx_v, buf0, buf1,
                      gsem0, gsem1, osem0, osem1):
        bufs = (buf0, buf1)
        gsems = (gsem0, gsem1)
        osems = (osem0, osem1)
        wid = lax.axis_index("s") * nc + lax.axis_index("c")
        base = pl.multiple_of(wid * rows_per_w, rows_per_w)
        pltpu.sync_copy(idx_hbm.at[pl.ds(base, rows_per_w)], idx_v)

        def start_gather(j, b):
            off = pl.multiple_of(j * _CHUNK, _CHUNK)
            pltpu.async_copy(
                table_hbm.at[idx_v.at[pl.ds(off, _CHUNK)]], bufs[b], gsems[b]
            )

        def start_out(j, b):
            off = pl.multiple_of(j * _CHUNK, _CHUNK)
            pltpu.async_copy(bufs[b], out_hbm.at[pl.ds(base + off, _CHUNK)],
                             osems[b])

        def wait_gather(b):
            pltpu.make_async_copy(
                out_hbm.at[pl.ds(base, _CHUNK)], bufs[b], gsems[b]).wait()

        def wait_out(b):
            pltpu.make_async_copy(
                bufs[b], out_hbm.at[pl.ds(base, _CHUNK)], osems[b]).wait()

        start_gather(0, 0)
        wait_gather(0)
        start_out(0, 0)
        start_gather(1, 1)

        def body(g, carry):
            for d in range(2):
                j = 2 * g + 1 + d
                b = (1 + d) % 2
                wait_gather(b)
                start_out(j, b)
                wait_out(1 - b)
                start_gather(j + 1, 1 - b)
            return carry

        lax.fori_loop(0, (n_chunks - 2) // 2, body, 0)

        wait_gather(1)
        start_out(n_chunks - 1, 1)
        wait_out(0)
        wait_out(1)

    return gather_kernel(table, idx)


def kernel(src, cbfv, W, b):
    batch, seq = src.shape
    d_model = W.shape[0]
    n_rows = batch * seq
    table = _project_table(cbfv, W, b)
    idx = src.reshape(-1).astype(jnp.int32)
    out = _sc_gather(table, idx, n_rows, d_model)
    return out.reshape(batch, seq, d_model)


# speedup vs baseline: 1.3561x; 1.3530x over previous
import functools

import jax
import jax.numpy as jnp
from jax import lax
from jax.experimental import pallas as pl
from jax.experimental.pallas import tpu as pltpu
from jax.experimental.pallas import tpu_sc as plsc


def _project_body(cbfv_ref, w_ref, b_ref, out_ref):
    acc = lax.dot_general(
        cbfv_ref[...], w_ref[...],
        dimension_numbers=(((1,), (1,)), ((), ())),
        preferred_element_type=jnp.float32,
    )
    out_ref[...] = acc + b_ref[...][None, :]


def _project_table(cbfv, W, b):
    vocab = cbfv.shape[0]
    d_model = W.shape[0]
    return pl.pallas_call(
        _project_body,
        out_shape=jax.ShapeDtypeStruct((vocab, d_model), jnp.float32),
    )(cbfv, W, b)


_CB = 4
_SEQ_PAD = 24


@functools.partial(jax.jit, static_argnums=(2, 3, 4))
def _sc_gather(table, idx_pad, batch, seq, d_model):
    try:
        info = plsc.get_sparse_core_info()
        nc, ns = info.num_cores, info.num_subcores
    except Exception:
        nc, ns = 2, 16
    nw = nc * ns
    assert batch % (nw * _CB) == 0
    b_per_w = batch // nw
    idx_per_w = b_per_w * _SEQ_PAD
    n_chunks = b_per_w // _CB
    assert n_chunks % 2 == 0 and n_chunks >= 4

    mesh = plsc.VectorSubcoreMesh(core_axis_name="c", subcore_axis_name="s")

    @functools.partial(
        pl.kernel,
        mesh=mesh,
        out_type=jax.ShapeDtypeStruct((batch, seq, d_model), jnp.float32),
        scratch_types=[
            pltpu.VMEM((idx_per_w,), jnp.int32),
            pltpu.VMEM((_CB, seq, d_model), jnp.float32),
            pltpu.VMEM((_CB, seq, d_model), jnp.float32),
            pltpu.SemaphoreType.DMA,
            pltpu.SemaphoreType.DMA,
            pltpu.SemaphoreType.DMA,
            pltpu.SemaphoreType.DMA,
        ],
    )
    def gather_kernel(table_hbm, idx_hbm, out_hbm, idx_v, buf0, buf1,
                      gsem0, gsem1, osem0, osem1):
        bufs = (buf0, buf1)
        gsems = (gsem0, gsem1)
        osems = (osem0, osem1)
        wid = lax.axis_index("s") * nc + lax.axis_index("c")
        ibase = pl.multiple_of(wid * idx_per_w, idx_per_w)
        bbase = pl.multiple_of(wid * b_per_w, b_per_w)
        pltpu.sync_copy(idx_hbm.at[pl.ds(ibase, idx_per_w)], idx_v)

        def start_gather(j, b):
            for k in range(_CB):
                off = pl.multiple_of((j * _CB + k) * _SEQ_PAD, _SEQ_PAD)
                pltpu.async_copy(
                    table_hbm.at[idx_v.at[pl.ds(off, seq)]],
                    bufs[b].at[k], gsems[b])

        def start_out(j, b):
            pltpu.async_copy(bufs[b],
                             out_hbm.at[pl.ds(bbase + j * _CB, _CB)], osems[b])

        def wait_gather(b):
            for k in range(_CB):
                pltpu.make_async_copy(
                    table_hbm.at[idx_v.at[pl.ds(0, seq)]],
                    bufs[b].at[k], gsems[b]).wait()

        def wait_out(b):
            pltpu.make_async_copy(
                bufs[b], out_hbm.at[pl.ds(bbase, _CB)], osems[b]).wait()

        start_gather(0, 0)
        wait_gather(0)
        start_out(0, 0)
        start_gather(1, 1)

        def body(g, carry):
            for d in range(2):
                j = 2 * g + 1 + d
                b = (1 + d) % 2
                wait_gather(b)
                start_out(j, b)
                wait_out(1 - b)
                start_gather(j + 1, 1 - b)
            return carry

        lax.fori_loop(0, (n_chunks - 2) // 2, body, 0)

        wait_gather(1)
        start_out(n_chunks - 1, 1)
        wait_out(0)
        wait_out(1)

    return gather_kernel(table, idx_pad)


def kernel(src, cbfv, W, b):
    batch, seq = src.shape
    d_model = W.shape[0]
    table = _project_table(cbfv, W, b)
    idx = src.astype(jnp.int32)
    idx_pad = jnp.pad(idx, ((0, 0), (0, _SEQ_PAD - seq))).reshape(-1)
    return _sc_gather(table, idx_pad, batch, seq, d_model)
